# baseline (speedup 1.0000x reference)
import jax
import jax.numpy as jnp
from jax.experimental import pallas as pl
from jax.experimental.pallas import tpu as pltpu

B, J, H, W = 64, 17, 64, 48
HW = H * W
TOPK_K = 8
BB = 8


def _ohkm_kernel(w_ref, p_ref, t_ref, out_ref, acc_ref):
    i = pl.program_id(0)
    p = p_ref[...]
    t = t_ref[...]
    w = w_ref[..., 0]
    d = p - t
    s = jnp.sum(d * d, axis=(2, 3))
    v = s * (w * w) * (0.5 / HW)

    col = jax.lax.broadcasted_iota(jnp.int32, (BB, J), 1)
    acc = jnp.zeros((BB,), jnp.float32)
    for _ in range(TOPK_K):
        m = jnp.max(v, axis=1)
        eq = v == m[:, None]
        idx = jnp.min(jnp.where(eq, col, J), axis=1)
        acc = acc + m
        v = jnp.where(col == idx[:, None], -jnp.inf, v)
    chunk = jnp.sum(acc)
    prev = jnp.where(i == 0, 0.0, acc_ref[0])
    total = prev + chunk
    acc_ref[0] = total

    @pl.when(i == pl.num_programs(0) - 1)
    def _finalize():
        out_ref[0, 0] = total * (1.0 / (TOPK_K * B))


def kernel(pred, target, target_weight):
    pred = jnp.swapaxes(pred, 2, 3)
    target = jnp.swapaxes(target, 2, 3)
    out = pl.pallas_call(
        _ohkm_kernel,
        grid=(B // BB,),
        in_specs=[
            pl.BlockSpec((BB, J, 1), lambda i: (i, 0, 0)),
            pl.BlockSpec((BB, J, W, H), lambda i: (i, 0, 0, 0)),
            pl.BlockSpec((BB, J, W, H), lambda i: (i, 0, 0, 0)),
        ],
        out_specs=pl.BlockSpec((1, 1), lambda i: (0, 0), memory_space=pltpu.SMEM),
        out_shape=jax.ShapeDtypeStruct((1, 1), jnp.float32),
        scratch_shapes=[pltpu.SMEM((1,), jnp.float32)],
    )(target_weight, pred, target)
    return out[0, 0]


# speedup vs baseline: 1.3254x; 1.3254x over previous
import jax
import jax.numpy as jnp
from jax.experimental import pallas as pl
from jax.experimental.pallas import tpu as pltpu

B, J, H, W = 64, 17, 64, 48
HW = H * W
TOPK_K = 8
BB = 8


def _top8_row_sums(v):
    rows = v.shape[0]
    col = jax.lax.broadcasted_iota(jnp.int32, (rows, J), 1)
    acc = jnp.zeros((rows,), jnp.float32)
    for _ in range(TOPK_K):
        m = jnp.max(v, axis=1)
        eq = v == m[:, None]
        idx = jnp.min(jnp.where(eq, col, J), axis=1)
        acc = acc + m
        v = jnp.where(col == idx[:, None], -jnp.inf, v)
    return acc


def _ohkm_kernel(w_ref, p_ref, t_ref, out_ref, loss_ref):
    i = pl.program_id(0)
    nsteps = pl.num_programs(0)
    p = p_ref[...]
    t = t_ref[...]
    w = w_ref[..., 0]
    d = p - t
    s = jnp.sum(d * d, axis=(2, 3))
    v_own = s * (w * w) * (0.5 / HW)

    @pl.when(i < nsteps - 1)
    def _store():
        loss_ref[pl.ds(i * BB, BB), :] = v_own

    @pl.when(i == nsteps - 1)
    def _finalize():
        head = _top8_row_sums(loss_ref[pl.ds(0, B - BB), :])
        tail = _top8_row_sums(v_own)
        out_ref[0, 0] = (jnp.sum(head) + jnp.sum(tail)) * (1.0 / (TOPK_K * B))


def kernel(pred, target, target_weight):
    pred = jnp.swapaxes(pred, 2, 3)
    target = jnp.swapaxes(target, 2, 3)
    out = pl.pallas_call(
        _ohkm_kernel,
        grid=(B // BB,),
        in_specs=[
            pl.BlockSpec((BB, J, 1), lambda i: (i, 0, 0)),
            pl.BlockSpec((BB, J, W, H), lambda i: (i, 0, 0, 0)),
            pl.BlockSpec((BB, J, W, H), lambda i: (i, 0, 0, 0)),
        ],
        out_specs=pl.BlockSpec((1, 1), lambda i: (0, 0), memory_space=pltpu.SMEM),
        out_shape=jax.ShapeDtypeStruct((1, 1), jnp.float32),
        scratch_shapes=[pltpu.VMEM((B, J), jnp.float32)],
    )(target_weight, pred, target)
    return out[0, 0]


# speedup vs baseline: 1.3641x; 1.0292x over previous
import jax
import jax.numpy as jnp
from jax.experimental import pallas as pl
from jax.experimental.pallas import tpu as pltpu

B, J, H, W = 64, 17, 64, 48
HW = H * W
TOPK_K = 8
BB = 8
INT_MIN = -2147483648


def _ohkm_kernel(w_ref, p_ref, t_ref, out_ref, loss_ref):
    i = pl.program_id(0)
    p = p_ref[...]
    t = t_ref[...]
    w = w_ref[..., 0]
    d = p - t
    s = jnp.sum(d * d, axis=(2, 3))
    loss_ref[pl.ds(i * BB, BB), :] = s * (w * w) * (0.5 / HW)

    @pl.when(i == pl.num_programs(0) - 1)
    def _finalize():
        v = loss_ref[...]
        col = jax.lax.broadcasted_iota(jnp.int32, (B, J), 1)
        bits = jax.lax.bitcast_convert_type(v, jnp.int32)
        key = jnp.bitwise_or(jnp.bitwise_and(bits, jnp.int32(~31)), 31 - col)
        acc = jnp.zeros((B,), jnp.float32)
        for _ in range(TOPK_K):
            m = jnp.max(key, axis=1)
            val = jax.lax.bitcast_convert_type(
                jnp.bitwise_and(m, jnp.int32(~31)), jnp.float32)
            acc = acc + val
            key = jnp.where(key == m[:, None], jnp.int32(INT_MIN), key)
        out_ref[0, 0] = jnp.sum(acc) * (1.0 / (TOPK_K * B))


def kernel(pred, target, target_weight):
    pred = jnp.swapaxes(pred, 2, 3)
    target = jnp.swapaxes(target, 2, 3)
    out = pl.pallas_call(
        _ohkm_kernel,
        grid=(B // BB,),
        in_specs=[
            pl.BlockSpec((BB, J, 1), lambda i: (i, 0, 0)),
            pl.BlockSpec((BB, J, W, H), lambda i: (i, 0, 0, 0)),
            pl.BlockSpec((BB, J, W, H), lambda i: (i, 0, 0, 0)),
        ],
        out_specs=pl.BlockSpec((1, 1), lambda i: (0, 0), memory_space=pltpu.SMEM),
        out_shape=jax.ShapeDtypeStruct((1, 1), jnp.float32),
        scratch_shapes=[pltpu.VMEM((B, J), jnp.float32)],
    )(target_weight, pred, target)
    return out[0, 0]
